# baseline (speedup 1.0000x reference)
import jax
import jax.numpy as jnp
from jax import lax
from jax.experimental import pallas as pl
from jax.experimental.pallas import tpu as pltpu

B, S, D, E = 2, 2048, 1024, 8
RC = 64
NRC = S // RC
NK = D // 128
CH = 256
NCH = S // CH
EPS2 = 1e-24


def _fused_kernel(s_ref, t_ref, t_any, o_any, acc_ref, rs_ref, buf,
                  sem_in0, sem_in1, sem_out0, sem_out1):
    e = pl.program_id(0)
    for b in range(B):
        @pl.when(e == 0)
        def _():
            for rc in range(NRC):
                r0 = rc * RC
                sn_acc = jnp.zeros((RC, 128), jnp.float32)
                for k in range(NK):
                    sfk = s_ref[b, r0:r0 + RC, k * 128:(k + 1) * 128]
                    sn_acc += sfk * sfk
                sn2 = jnp.sum(sn_acc, axis=1, keepdims=True)
                rs_ref[b, r0:r0 + RC, :] = lax.rsqrt(jnp.maximum(sn2, EPS2))

        part = jnp.zeros((1, 1), jnp.float32)
        for rc in range(NRC):
            r0 = rc * RC
            dot_acc = jnp.zeros((RC, 128), jnp.float32)
            tn_acc = jnp.zeros((RC, 128), jnp.float32)
            for k in range(NK):
                sfk = s_ref[b, r0:r0 + RC, k * 128:(k + 1) * 128]
                tfk = t_ref[0, b, r0:r0 + RC, k * 128:(k + 1) * 128]
                dot_acc += sfk * tfk
                tn_acc += tfk * tfk
            dot = jnp.sum(dot_acc, axis=1, keepdims=True)
            tn2 = jnp.sum(tn_acc, axis=1, keepdims=True)
            rt = lax.rsqrt(jnp.maximum(tn2, EPS2))
            w = dot * rt * rs_ref[b, r0:r0 + RC, :]
            part += jnp.sum(w, axis=0, keepdims=True)
        acc_ref[b, pl.ds(e, 1), :] = part

    @pl.when(e == E - 1)
    def _():
        idx = [jnp.argmax(acc_ref[b][:, 0], axis=0) for b in range(B)]
        sems_in = [sem_in0, sem_in1]
        sems_out = [sem_out0, sem_out1]

        def copy_in(k):
            b, c = divmod(k, NCH)
            s0 = c * CH
            return pltpu.make_async_copy(
                t_any.at[idx[b], b, pl.ds(s0, CH)], buf.at[k % 2],
                sems_in[k % 2])

        def copy_out(k):
            b, c = divmod(k, NCH)
            s0 = c * CH
            return pltpu.make_async_copy(
                buf.at[k % 2], o_any.at[b, pl.ds(s0, CH)], sems_out[k % 2])

        copy_in(0).start()
        copy_in(1).start()
        for k in range(B * NCH):
            copy_in(k).wait()
            copy_out(k).start()
            copy_out(k).wait()
            if k + 2 < B * NCH:
                copy_in(k + 2).start()


@jax.jit
def kernel(student_features, teacher_features):
    return pl.pallas_call(
        _fused_kernel,
        grid=(E,),
        in_specs=[
            pl.BlockSpec((B, S, D), lambda e: (0, 0, 0)),
            pl.BlockSpec((1, B, S, D), lambda e: (e, 0, 0, 0)),
            pl.BlockSpec(memory_space=pl.ANY),
        ],
        out_specs=pl.BlockSpec(memory_space=pl.ANY),
        out_shape=jax.ShapeDtypeStruct((B, S, D), jnp.float32),
        scratch_shapes=[
            pltpu.VMEM((B, E, 1), jnp.float32),
            pltpu.VMEM((B, S, 1), jnp.float32),
            pltpu.VMEM((2, CH, D), jnp.float32),
            pltpu.SemaphoreType.DMA,
            pltpu.SemaphoreType.DMA,
            pltpu.SemaphoreType.DMA,
            pltpu.SemaphoreType.DMA,
        ],
        compiler_params=pltpu.CompilerParams(
            dimension_semantics=("arbitrary",),
        ),
    )(student_features, teacher_features, teacher_features)


# speedup vs baseline: 1.1662x; 1.1662x over previous
import jax
import jax.numpy as jnp
from jax import lax
from jax.experimental import pallas as pl
from jax.experimental.pallas import tpu as pltpu

B, S, D, E = 2, 2048, 1024, 8
S_BLK = 2048
NS = S // S_BLK
C_BLK = 2048
NCB = S // C_BLK
RC = 64
NRC = S_BLK // RC
NK = D // 128
EPS2 = 1e-24


def _sims_kernel(s_ref, t_ref, idx_ref, acc_ref, rs_ref):
    s = pl.program_id(0)
    e = pl.program_id(1)
    for b in range(B):
        @pl.when(e == 0)
        def _():
            for rc in range(NRC):
                r0 = rc * RC
                sn_acc = jnp.zeros((RC, 128), jnp.float32)
                for k in range(NK):
                    sfk = s_ref[b, r0:r0 + RC, k * 128:(k + 1) * 128]
                    sn_acc += sfk * sfk
                sn2 = jnp.sum(sn_acc, axis=1, keepdims=True)
                rs_ref[b, r0:r0 + RC, :] = lax.rsqrt(jnp.maximum(sn2, EPS2))

        part = jnp.zeros((1, 1), jnp.float32)
        for rc in range(NRC):
            r0 = rc * RC
            dot_acc = jnp.zeros((RC, 128), jnp.float32)
            tn_acc = jnp.zeros((RC, 128), jnp.float32)
            for k in range(NK):
                sfk = s_ref[b, r0:r0 + RC, k * 128:(k + 1) * 128]
                tfk = t_ref[0, b, r0:r0 + RC, k * 128:(k + 1) * 128]
                dot_acc += sfk * tfk
                tn_acc += tfk * tfk
            dot = jnp.sum(dot_acc, axis=1, keepdims=True)
            tn2 = jnp.sum(tn_acc, axis=1, keepdims=True)
            rt = lax.rsqrt(jnp.maximum(tn2, EPS2))
            w = dot * rt * rs_ref[b, r0:r0 + RC, :]
            part += jnp.sum(w, axis=0, keepdims=True)
        prev = acc_ref[b, pl.ds(e, 1), :]
        acc_ref[b, pl.ds(e, 1), :] = jnp.where(s == 0, part, prev + part)

    @pl.when((s == NS - 1) & (e == E - 1))
    def _():
        for b in range(B):
            sims = acc_ref[b]
            idx_ref[b] = jnp.argmax(sims[:, 0], axis=0).astype(jnp.int32)


def _copy_kernel(idx_ref, t_ref, o_ref):
    del idx_ref
    o_ref[...] = t_ref[0]


@jax.jit
def kernel(student_features, teacher_features):
    idx = pl.pallas_call(
        _sims_kernel,
        grid=(NS, E),
        in_specs=[
            pl.BlockSpec((B, S_BLK, D), lambda s, e: (0, s, 0)),
            pl.BlockSpec((1, B, S_BLK, D), lambda s, e: (e, 0, s, 0)),
        ],
        out_specs=pl.BlockSpec(memory_space=pltpu.SMEM),
        out_shape=jax.ShapeDtypeStruct((B,), jnp.int32),
        scratch_shapes=[
            pltpu.VMEM((B, E, 1), jnp.float32),
            pltpu.VMEM((B, S_BLK, 1), jnp.float32),
        ],
        compiler_params=pltpu.CompilerParams(
            dimension_semantics=("arbitrary", "arbitrary"),
        ),
    )(student_features, teacher_features)

    grid_spec = pltpu.PrefetchScalarGridSpec(
        num_scalar_prefetch=1,
        grid=(B, NCB),
        in_specs=[
            pl.BlockSpec((1, 1, C_BLK, D), lambda b, s, idx_ref: (idx_ref[b], b, s, 0)),
        ],
        out_specs=pl.BlockSpec((1, C_BLK, D), lambda b, s, idx_ref: (b, s, 0)),
    )
    out = pl.pallas_call(
        _copy_kernel,
        grid_spec=grid_spec,
        out_shape=jax.ShapeDtypeStruct((B, S, D), jnp.float32),
    )(idx, teacher_features)
    return out
